# baseline (speedup 1.0000x reference)
import jax
import jax.numpy as jnp
from jax import lax
from jax.experimental import pallas as pl
from jax.experimental.pallas import tpu as pltpu
from jax.experimental.pallas import tpu_sc as plsc


def _scs_body(x_hbm, w_hbm, b_hbm, tok_hbm, out_hbm, x_s, w_s, b_s, tok_s,
              out_s, sem):
    c1 = pltpu.async_copy(x_hbm, x_s, sem)
    c2 = pltpu.async_copy(w_hbm, w_s, sem)
    c3 = pltpu.async_copy(b_hbm, b_s, sem)
    c4 = pltpu.async_copy(tok_hbm, tok_s, sem)
    c1.wait()
    c2.wait()
    c3.wait()
    c4.wait()
    for j in range(16):
        acc0 = b_s[j]
        acc1 = b_s[j]
        for k in range(8):
            w_jk = w_s[j * 8 + k]
            acc0 = acc0 + tok_s[k] * w_jk
            acc1 = acc1 + x_s[8 + k] * w_jk
        out_s[j] = acc0
        out_s[16 + j] = acc1
    pltpu.sync_copy(out_s, out_hbm)


def kernel(x, W, b, token):
    mesh = plsc.ScalarSubcoreMesh(axis_name="c", num_cores=1)
    x_flat = x.reshape(-1)
    w_flat = W.reshape(-1)
    f = pl.kernel(
        _scs_body,
        out_type=jax.ShapeDtypeStruct((32,), jnp.float32),
        mesh=mesh,
        scratch_types=[
            pltpu.SMEM((16,), jnp.float32),
            pltpu.SMEM((128,), jnp.float32),
            pltpu.SMEM((16,), jnp.float32),
            pltpu.SMEM((8,), jnp.float32),
            pltpu.SMEM((32,), jnp.float32),
            pltpu.SemaphoreType.DMA,
        ],
    )
    return f(x_flat, w_flat, b, token).reshape(2, 16)


# speedup vs baseline: 1.0240x; 1.0240x over previous
import jax
import jax.numpy as jnp
from jax import lax
from jax.experimental import pallas as pl
from jax.experimental.pallas import tpu as pltpu
from jax.experimental.pallas import tpu_sc as plsc


def _sc_body(p_hbm, out_hbm, p_v, out_v):
    sid = lax.axis_index("s")

    @pl.when(sid == 0)
    def _():
        pltpu.sync_copy(p_hbm, p_v)
        xvec = p_v[pl.ds(0, 16)]
        bvec = p_v[pl.ds(16, 16)]
        tokvec = p_v[pl.ds(32, 16)]
        acc0 = bvec
        acc1 = bvec
        for k in range(8):
            col = p_v[pl.ds(48 + 16 * k, 16)]
            acc0 = acc0 + tokvec[k] * col
            acc1 = acc1 + xvec[8 + k] * col
        out_v[0, :] = acc0
        out_v[1, :] = acc1
        pltpu.sync_copy(out_v, out_hbm)


def kernel(x, W, b, token):
    mesh = plsc.VectorSubcoreMesh(
        core_axis_name="c", subcore_axis_name="s", num_cores=1
    )
    packed = jnp.concatenate(
        [x.reshape(-1), b, token, jnp.zeros((8,), jnp.float32), W.T.reshape(-1)]
    )
    f = pl.kernel(
        _sc_body,
        out_type=jax.ShapeDtypeStruct((2, 16), jnp.float32),
        mesh=mesh,
        scratch_types=[
            pltpu.VMEM((176,), jnp.float32),
            pltpu.VMEM((2, 16), jnp.float32),
        ],
    )
    return f(packed)
